# baseline (speedup 1.0000x reference)
import functools

import jax
import jax.numpy as jnp
from jax import lax
from jax.experimental import pallas as pl
from jax.experimental.pallas import tpu as pltpu
from jax.experimental.pallas import tpu_sc as plsc

KE = 14.399645351950548

NC = 2
NS = 16
NW = NC * NS
LANES = 16
CHUNK = 128
ROWS = 8
BLK = ROWS * CHUNK
ZCH = 1024


def _f16bits(u):
    rnd = jnp.bitwise_and(lax.shift_right_logical(u, 13), 1)
    return lax.shift_right_logical(u + 0xFFF + rnd - 0x38000000, 13)


def _table_body(zf_ref, ap_ref, out_ref):
    zf = zf_ref[...]
    z = jnp.exp(ap_ref[0] * jnp.log(zf))
    hz = _f16bits(lax.bitcast_convert_type(z, jnp.int32))
    hq = _f16bits(lax.bitcast_convert_type(zf, jnp.int32))
    out_ref[...] = jnp.bitwise_or(lax.shift_left(hz, 16), hq)


def _sum_body(p_ref, o_ref):
    o_ref[...] = p_ref[0] + p_ref[1]


def _dec(h):
    return lax.bitcast_convert_type(lax.shift_left(h, 13) + 0x38000000,
                                    jnp.float32)


def _sc_body(nblk_base, nblk_extra, nchunks,
             tab_h, par_h, ii_h, jj_h, rr_h, cc_h, out_h,
             tab_v, par_v, zbuf, ii_v, jj_v, r_v, cu_v, ct_v, e_sh,
             ii_sem, in_sem, sc_sem, aux_sem):
    cid = lax.axis_index("c")
    sid = lax.axis_index("s")
    wid = sid * NC + cid

    nb = nblk_base + (wid < nblk_extra).astype(jnp.int32)
    sb = wid * nblk_base + jnp.minimum(wid, nblk_extra)

    tcp = pltpu.async_copy(tab_h, tab_v, aux_sem)
    pcp = pltpu.async_copy(par_h, par_v, aux_sem)

    zero = jnp.zeros((LANES,), jnp.float32)

    def _zrow(r, c):
        for q in range(8):
            zbuf[pl.ds(r * CHUNK + q * LANES, LANES)] = zero
        return c

    lax.fori_loop(0, ZCH // CHUNK, _zrow, 0)
    for k in range((nchunks + NS - 1) // NS):
        ci = sid + k * NS

        @pl.when(ci < nchunks)
        def _():
            pltpu.sync_copy(zbuf, e_sh.at[pl.ds(ci * ZCH, ZCH)])

    tcp.wait()
    pcp.wait()
    plsc.subcore_barrier()

    nq1 = par_v[0]
    nq2 = par_v[1]
    nq3 = par_v[2]
    nq4 = par_v[3]
    cc1 = par_v[4]
    cc2 = par_v[5]
    cc3 = par_v[6]
    cc4 = par_v[7]

    def _issue_in(b):
        row0 = (sb + b) * ROWS
        s3 = lax.rem(b, 3)
        s2 = lax.rem(b, 2)
        pltpu.async_copy(ii_h.at[pl.ds(row0, ROWS)], ii_v.at[s3],
                         ii_sem.at[s3])
        pltpu.async_copy(jj_h.at[pl.ds(row0, ROWS)], jj_v.at[s2],
                         in_sem.at[s2])
        pltpu.async_copy(rr_h.at[pl.ds(row0, ROWS)], r_v.at[s2],
                         in_sem.at[s2])
        pltpu.async_copy(cc_h.at[pl.ds(row0, ROWS)], cu_v.at[s2],
                         in_sem.at[s2])

    def _wait_in(b):
        row0 = (sb + b) * ROWS
        s3 = lax.rem(b, 3)
        s2 = lax.rem(b, 2)
        pltpu.make_async_copy(ii_h.at[pl.ds(row0, ROWS)], ii_v.at[s3],
                              ii_sem.at[s3]).wait()
        pltpu.make_async_copy(jj_h.at[pl.ds(row0, ROWS)], jj_v.at[s2],
                              in_sem.at[s2]).wait()
        pltpu.make_async_copy(rr_h.at[pl.ds(row0, ROWS)], r_v.at[s2],
                              in_sem.at[s2]).wait()
        pltpu.make_async_copy(cc_h.at[pl.ds(row0, ROWS)], cu_v.at[s2],
                              in_sem.at[s2]).wait()

    def _drain_sc(b):
        s3 = lax.rem(b, 3)
        for c in range(ROWS):
            pltpu.make_async_copy(ct_v.at[s3, c],
                                  e_sh.at[ii_v.at[s3, c]],
                                  sc_sem.at[s3]).wait()

    _issue_in(0)

    def _block(b, carry):
        s3 = lax.rem(b, 3)
        s2 = lax.rem(b, 2)

        @pl.when(b >= 2)
        def _():
            _drain_sc(b - 2)

        @pl.when(b + 1 < nb)
        def _():
            _issue_in(b + 1)

        _wait_in(b)

        @plsc.parallel_loop(0, ROWS * 8, unroll=8)
        def _row(v):
            c = lax.shift_right_logical(v, 3)
            q = jnp.bitwise_and(v, 7)
            if True:
                sl = pl.ds(q * LANES, LANES)
                iiv = ii_v[s3, c, sl]
                jjv = jj_v[s2, c, sl]
                wi = plsc.load_gather(tab_v, [iiv])
                wj = plsc.load_gather(tab_v, [jjv])
                zi = _dec(lax.shift_right_logical(wi, 16))
                zj = _dec(lax.shift_right_logical(wj, 16))
                qi = _dec(jnp.bitwise_and(wi, 0xFFFF))
                qj = _dec(jnp.bitwise_and(wj, 0xFFFF))
                rv = r_v[s2, c, sl]
                cuv = cu_v[s2, c, sl]
                s = (zi + zj) * rv
                f = (cc1 * jnp.exp(nq1 * s) + cc2 * jnp.exp(nq2 * s)
                     + cc3 * jnp.exp(nq3 * s) + cc4 * jnp.exp(nq4 * s))
                ct_v[s3, c, sl] = f * (qi * qj) * (cuv / rv)

        for c in range(ROWS):
            pltpu.async_copy(ct_v.at[s3, c], e_sh.at[ii_v.at[s3, c]],
                             sc_sem.at[s3], add=True)
        return carry

    lax.fori_loop(0, nb, _block, 0)

    @pl.when(nb >= 2)
    def _():
        _drain_sc(nb - 2)

    @pl.when(nb >= 1)
    def _():
        _drain_sc(nb - 1)

    plsc.subcore_barrier()

    @pl.when(sid == 0)
    def _():
        pltpu.sync_copy(e_sh, out_h.at[cid])


def kernel(N, Zf, rij, cutoff_values, idx_i, idx_j,
           _adiv, _apow, _c1, _c2, _c3, _c4, _a1, _a2, _a3, _a4):
    f32 = jnp.float32
    sp = jax.nn.softplus
    n = Zf.shape[0]
    p = rij.shape[0]
    npad = ((n + ZCH - 1) // ZCH) * ZCH
    nchunks = npad // ZCH
    nrows = p // CHUNK
    tblk = nrows // ROWS
    nblk_base, nblk_extra = divmod(tblk, NW)

    adiv = sp(_adiv)[0]
    apow = sp(_apow)[0]
    aa = [sp(x)[0] for x in (_a1, _a2, _a3, _a4)]
    cc = [sp(x)[0] for x in (_c1, _c2, _c3, _c4)]
    csum = cc[0] + cc[1] + cc[2] + cc[3]
    keh = KE / 2.0
    par = jnp.stack([-(aa[0] * adiv), -(aa[1] * adiv),
                     -(aa[2] * adiv), -(aa[3] * adiv),
                     keh * cc[0] / csum, keh * cc[1] / csum,
                     keh * cc[2] / csum, keh * cc[3] / csum]).astype(f32)
    par = jnp.broadcast_to(par[:, None], (8, LANES))

    zf_pad = jnp.concatenate([Zf.astype(f32),
                              jnp.ones((npad - n,), f32)])
    zf2 = zf_pad.reshape(npad // CHUNK, CHUNK)
    ap1 = jnp.reshape(apow, (1,)).astype(f32)

    tab2 = pl.pallas_call(
        _table_body,
        out_shape=jax.ShapeDtypeStruct((npad // CHUNK, CHUNK), jnp.int32),
        in_specs=[pl.BlockSpec(memory_space=pltpu.VMEM),
                  pl.BlockSpec(memory_space=pltpu.SMEM)],
        out_specs=pl.BlockSpec(memory_space=pltpu.VMEM),
    )(zf2, ap1)
    tab = tab2.reshape(npad)

    ii2 = idx_i.astype(jnp.int32).reshape(nrows, CHUNK)
    jj2 = idx_j.astype(jnp.int32).reshape(nrows, CHUNK)
    rr2 = rij.astype(f32).reshape(nrows, CHUNK)
    cc2 = cutoff_values.astype(f32).reshape(nrows, CHUNK)

    mesh = plsc.VectorSubcoreMesh(core_axis_name="c", subcore_axis_name="s")
    parts = pl.kernel(
        functools.partial(_sc_body, nblk_base, nblk_extra, nchunks),
        out_type=jax.ShapeDtypeStruct((NC, npad), f32),
        mesh=mesh,
        compiler_params=pltpu.CompilerParams(needs_layout_passes=False),
        scratch_types=[
            pltpu.VMEM((npad,), jnp.int32),
            pltpu.VMEM((8, LANES), f32),
            pltpu.VMEM((ZCH,), f32),
            pltpu.VMEM((3, ROWS, CHUNK), jnp.int32),
            pltpu.VMEM((2, ROWS, CHUNK), jnp.int32),
            pltpu.VMEM((2, ROWS, CHUNK), f32),
            pltpu.VMEM((2, ROWS, CHUNK), f32),
            pltpu.VMEM((3, ROWS, CHUNK), f32),
            pltpu.VMEM_SHARED((npad,), f32),
            pltpu.SemaphoreType.DMA((3,)),
            pltpu.SemaphoreType.DMA((2,)),
            pltpu.SemaphoreType.DMA((3,)),
            pltpu.SemaphoreType.DMA,
        ],
    )(tab, par, ii2, jj2, rr2, cc2)

    e2 = pl.pallas_call(
        _sum_body,
        out_shape=jax.ShapeDtypeStruct((npad // CHUNK, CHUNK), f32),
        in_specs=[pl.BlockSpec(memory_space=pltpu.VMEM)],
        out_specs=pl.BlockSpec(memory_space=pltpu.VMEM),
    )(parts.reshape(NC, npad // CHUNK, CHUNK))
    return e2.reshape(npad)[:n]


# speedup vs baseline: 1.0270x; 1.0270x over previous
import functools

import jax
import jax.numpy as jnp
from jax import lax
from jax.experimental import pallas as pl
from jax.experimental.pallas import tpu as pltpu
from jax.experimental.pallas import tpu_sc as plsc

KE = 14.399645351950548

NC = 2
NS = 16
NW = NC * NS
LANES = 16
CHUNK = 128
ROWS = 8
BLK = ROWS * CHUNK
ZCH = 1024


def _f16bits(u):
    rnd = jnp.bitwise_and(lax.shift_right_logical(u, 13), 1)
    return lax.shift_right_logical(u + 0xFFF + rnd - 0x38000000, 13)


def _table_body(zf_ref, ap_ref, out_ref):
    zf = zf_ref[...]
    z = jnp.exp(ap_ref[0] * jnp.log(zf))
    hz = _f16bits(lax.bitcast_convert_type(z, jnp.int32))
    hq = _f16bits(lax.bitcast_convert_type(zf, jnp.int32))
    out_ref[...] = jnp.bitwise_or(lax.shift_left(hz, 16), hq)


def _sum_body(p_ref, o_ref):
    o_ref[...] = p_ref[0] + p_ref[1]


def _dec(h):
    return lax.bitcast_convert_type(lax.shift_left(h, 13) + 0x38000000,
                                    jnp.float32)


def _sc_body(nblk_base, nblk_extra, nchunks,
             tab_h, par_h, ii_h, jj_h, rr_h, cc_h, out_h,
             tab_v, par_v, zbuf, ii_v, jj_v, r_v, cu_v, ct_v, e_sh,
             ii_sem, in_sem, sc_sem, aux_sem):
    cid = lax.axis_index("c")
    sid = lax.axis_index("s")
    wid = sid * NC + cid

    nb = nblk_base + (wid < nblk_extra).astype(jnp.int32)
    sb = wid * nblk_base + jnp.minimum(wid, nblk_extra)

    tcp = pltpu.async_copy(tab_h, tab_v, aux_sem)
    pcp = pltpu.async_copy(par_h, par_v, aux_sem)

    zero = jnp.zeros((LANES,), jnp.float32)

    def _zrow(r, c):
        for q in range(8):
            zbuf[pl.ds(r * CHUNK + q * LANES, LANES)] = zero
        return c

    lax.fori_loop(0, ZCH // CHUNK, _zrow, 0)
    for k in range((nchunks + NS - 1) // NS):
        ci = sid + k * NS

        @pl.when(ci < nchunks)
        def _():
            pltpu.sync_copy(zbuf, e_sh.at[pl.ds(ci * ZCH, ZCH)])

    tcp.wait()
    pcp.wait()
    plsc.subcore_barrier()

    nq1 = par_v[0]
    nq2 = par_v[1]
    nq3 = par_v[2]
    nq4 = par_v[3]
    cc1 = par_v[4]
    cc2 = par_v[5]
    cc3 = par_v[6]
    cc4 = par_v[7]

    def _issue_in(b):
        row0 = (sb + b) * ROWS
        s3 = lax.rem(b, 3)
        s2 = lax.rem(b, 2)
        pltpu.async_copy(ii_h.at[pl.ds(row0, ROWS)], ii_v.at[s3],
                         ii_sem.at[s3])
        pltpu.async_copy(jj_h.at[pl.ds(row0, ROWS)], jj_v.at[s2],
                         in_sem.at[s2])
        pltpu.async_copy(rr_h.at[pl.ds(row0, ROWS)], r_v.at[s2],
                         in_sem.at[s2])
        pltpu.async_copy(cc_h.at[pl.ds(row0, ROWS)], cu_v.at[s2],
                         in_sem.at[s2])

    def _wait_in(b):
        row0 = (sb + b) * ROWS
        s3 = lax.rem(b, 3)
        s2 = lax.rem(b, 2)
        pltpu.make_async_copy(ii_h.at[pl.ds(row0, ROWS)], ii_v.at[s3],
                              ii_sem.at[s3]).wait()
        pltpu.make_async_copy(jj_h.at[pl.ds(row0, ROWS)], jj_v.at[s2],
                              in_sem.at[s2]).wait()
        pltpu.make_async_copy(rr_h.at[pl.ds(row0, ROWS)], r_v.at[s2],
                              in_sem.at[s2]).wait()
        pltpu.make_async_copy(cc_h.at[pl.ds(row0, ROWS)], cu_v.at[s2],
                              in_sem.at[s2]).wait()

    def _drain_sc(b):
        s3 = lax.rem(b, 3)
        for c in range(ROWS):
            pltpu.make_async_copy(ct_v.at[s3, c],
                                  e_sh.at[ii_v.at[s3, c]],
                                  sc_sem.at[s3]).wait()

    _issue_in(0)

    def _block(b, carry):
        s3 = lax.rem(b, 3)
        s2 = lax.rem(b, 2)

        @pl.when(b >= 2)
        def _():
            _drain_sc(b - 2)

        @pl.when(b + 1 < nb)
        def _():
            _issue_in(b + 1)

        _wait_in(b)

        @plsc.parallel_loop(0, ROWS * 8, unroll=2)
        def _row(v):
            c = lax.shift_right_logical(v, 3)
            q = jnp.bitwise_and(v, 7)
            if True:
                sl = pl.ds(q * LANES, LANES)
                iiv = ii_v[s3, c, sl]
                jjv = jj_v[s2, c, sl]
                wi = plsc.load_gather(tab_v, [iiv])
                wj = plsc.load_gather(tab_v, [jjv])
                zi = _dec(lax.shift_right_logical(wi, 16))
                zj = _dec(lax.shift_right_logical(wj, 16))
                qi = _dec(jnp.bitwise_and(wi, 0xFFFF))
                qj = _dec(jnp.bitwise_and(wj, 0xFFFF))
                rv = r_v[s2, c, sl]
                cuv = cu_v[s2, c, sl]
                s = (zi + zj) * rv
                f = (cc1 * jnp.exp(nq1 * s) + cc2 * jnp.exp(nq2 * s)
                     + cc3 * jnp.exp(nq3 * s) + cc4 * jnp.exp(nq4 * s))
                ct_v[s3, c, sl] = f * (qi * qj) * (cuv / rv)

        for c in range(ROWS):
            pltpu.async_copy(ct_v.at[s3, c], e_sh.at[ii_v.at[s3, c]],
                             sc_sem.at[s3], add=True)
        return carry

    lax.fori_loop(0, nb, _block, 0)

    @pl.when(nb >= 2)
    def _():
        _drain_sc(nb - 2)

    @pl.when(nb >= 1)
    def _():
        _drain_sc(nb - 1)

    plsc.subcore_barrier()

    @pl.when(sid == 0)
    def _():
        pltpu.sync_copy(e_sh, out_h.at[cid])


def kernel(N, Zf, rij, cutoff_values, idx_i, idx_j,
           _adiv, _apow, _c1, _c2, _c3, _c4, _a1, _a2, _a3, _a4):
    f32 = jnp.float32
    sp = jax.nn.softplus
    n = Zf.shape[0]
    p = rij.shape[0]
    npad = ((n + ZCH - 1) // ZCH) * ZCH
    nchunks = npad // ZCH
    nrows = p // CHUNK
    tblk = nrows // ROWS
    nblk_base, nblk_extra = divmod(tblk, NW)

    adiv = sp(_adiv)[0]
    apow = sp(_apow)[0]
    aa = [sp(x)[0] for x in (_a1, _a2, _a3, _a4)]
    cc = [sp(x)[0] for x in (_c1, _c2, _c3, _c4)]
    csum = cc[0] + cc[1] + cc[2] + cc[3]
    keh = KE / 2.0
    par = jnp.stack([-(aa[0] * adiv), -(aa[1] * adiv),
                     -(aa[2] * adiv), -(aa[3] * adiv),
                     keh * cc[0] / csum, keh * cc[1] / csum,
                     keh * cc[2] / csum, keh * cc[3] / csum]).astype(f32)
    par = jnp.broadcast_to(par[:, None], (8, LANES))

    zf_pad = jnp.concatenate([Zf.astype(f32),
                              jnp.ones((npad - n,), f32)])
    zf2 = zf_pad.reshape(npad // CHUNK, CHUNK)
    ap1 = jnp.reshape(apow, (1,)).astype(f32)

    tab2 = pl.pallas_call(
        _table_body,
        out_shape=jax.ShapeDtypeStruct((npad // CHUNK, CHUNK), jnp.int32),
        in_specs=[pl.BlockSpec(memory_space=pltpu.VMEM),
                  pl.BlockSpec(memory_space=pltpu.SMEM)],
        out_specs=pl.BlockSpec(memory_space=pltpu.VMEM),
    )(zf2, ap1)
    tab = tab2.reshape(npad)

    ii2 = idx_i.astype(jnp.int32).reshape(nrows, CHUNK)
    jj2 = idx_j.astype(jnp.int32).reshape(nrows, CHUNK)
    rr2 = rij.astype(f32).reshape(nrows, CHUNK)
    cc2 = cutoff_values.astype(f32).reshape(nrows, CHUNK)

    mesh = plsc.VectorSubcoreMesh(core_axis_name="c", subcore_axis_name="s")
    parts = pl.kernel(
        functools.partial(_sc_body, nblk_base, nblk_extra, nchunks),
        out_type=jax.ShapeDtypeStruct((NC, npad), f32),
        mesh=mesh,
        compiler_params=pltpu.CompilerParams(needs_layout_passes=False),
        scratch_types=[
            pltpu.VMEM((npad,), jnp.int32),
            pltpu.VMEM((8, LANES), f32),
            pltpu.VMEM((ZCH,), f32),
            pltpu.VMEM((3, ROWS, CHUNK), jnp.int32),
            pltpu.VMEM((2, ROWS, CHUNK), jnp.int32),
            pltpu.VMEM((2, ROWS, CHUNK), f32),
            pltpu.VMEM((2, ROWS, CHUNK), f32),
            pltpu.VMEM((3, ROWS, CHUNK), f32),
            pltpu.VMEM_SHARED((npad,), f32),
            pltpu.SemaphoreType.DMA((3,)),
            pltpu.SemaphoreType.DMA((2,)),
            pltpu.SemaphoreType.DMA((3,)),
            pltpu.SemaphoreType.DMA,
        ],
    )(tab, par, ii2, jj2, rr2, cc2)

    e2 = pl.pallas_call(
        _sum_body,
        out_shape=jax.ShapeDtypeStruct((npad // CHUNK, CHUNK), f32),
        in_specs=[pl.BlockSpec(memory_space=pltpu.VMEM)],
        out_specs=pl.BlockSpec(memory_space=pltpu.VMEM),
    )(parts.reshape(NC, npad // CHUNK, CHUNK))
    return e2.reshape(npad)[:n]


# speedup vs baseline: 1.1436x; 1.1136x over previous
import functools

import jax
import jax.numpy as jnp
from jax import lax
from jax.experimental import pallas as pl
from jax.experimental.pallas import tpu as pltpu
from jax.experimental.pallas import tpu_sc as plsc

KE = 14.399645351950548

NC = 2
NS = 16
NW = NC * NS
LANES = 16
CHUNK = 128
ROWS = 8
BLK = ROWS * CHUNK
ZCH = 1024


def _f16bits(u):
    rnd = jnp.bitwise_and(lax.shift_right_logical(u, 13), 1)
    return lax.shift_right_logical(u + 0xFFF + rnd - 0x38000000, 13)


def _table_body(zf_ref, ap_ref, out_ref):
    zf = zf_ref[...]
    z = jnp.exp(ap_ref[0] * jnp.log(zf))
    hz = _f16bits(lax.bitcast_convert_type(z, jnp.int32))
    hq = _f16bits(lax.bitcast_convert_type(zf, jnp.int32))
    out_ref[...] = jnp.bitwise_or(lax.shift_left(hz, 16), hq)


def _sum_body(p_ref, o_ref):
    o_ref[...] = p_ref[0] + p_ref[1]


def _dec(h):
    return lax.bitcast_convert_type(lax.shift_left(h, 13) + 0x38000000,
                                    jnp.float32)


def _sc_body(nblk_base, nblk_extra, nchunks,
             tab_h, par_h, ii_h, jj_h, rr_h, cc_h, out_h,
             tab_v, par_v, zbuf, ii_v, jj_v, r_v, cu_v, ct_v, e_sh,
             ii_sem, in_sem, sc_sem, aux_sem):
    cid = lax.axis_index("c")
    sid = lax.axis_index("s")
    wid = sid * NC + cid

    nb = nblk_base + (wid < nblk_extra).astype(jnp.int32)
    sb = wid * nblk_base + jnp.minimum(wid, nblk_extra)

    tcp = pltpu.async_copy(tab_h, tab_v, aux_sem)
    pcp = pltpu.async_copy(par_h, par_v, aux_sem)

    zero = jnp.zeros((LANES,), jnp.float32)

    def _zrow(r, c):
        for q in range(8):
            zbuf[pl.ds(r * CHUNK + q * LANES, LANES)] = zero
        return c

    lax.fori_loop(0, ZCH // CHUNK, _zrow, 0)
    for k in range((nchunks + NS - 1) // NS):
        ci = sid + k * NS

        @pl.when(ci < nchunks)
        def _():
            pltpu.sync_copy(zbuf, e_sh.at[pl.ds(ci * ZCH, ZCH)])

    tcp.wait()
    pcp.wait()
    plsc.subcore_barrier()

    nq1 = par_v[0]
    nq2 = par_v[1]
    nq3 = par_v[2]
    nq4 = par_v[3]
    cc1 = par_v[4]
    cc2 = par_v[5]
    cc3 = par_v[6]
    cc4 = par_v[7]

    def _issue_in(b):
        row0 = (sb + b) * ROWS
        s5 = lax.rem(b, 5)
        s4 = lax.rem(b, 4)
        pltpu.async_copy(ii_h.at[pl.ds(row0, ROWS)], ii_v.at[s5],
                         ii_sem.at[s5])
        pltpu.async_copy(jj_h.at[pl.ds(row0, ROWS)], jj_v.at[s4],
                         in_sem.at[s4])
        pltpu.async_copy(rr_h.at[pl.ds(row0, ROWS)], r_v.at[s4],
                         in_sem.at[s4])
        pltpu.async_copy(cc_h.at[pl.ds(row0, ROWS)], cu_v.at[s4],
                         in_sem.at[s4])

    def _wait_in(b):
        row0 = (sb + b) * ROWS
        s5 = lax.rem(b, 5)
        s4 = lax.rem(b, 4)
        pltpu.make_async_copy(ii_h.at[pl.ds(row0, ROWS)], ii_v.at[s5],
                              ii_sem.at[s5]).wait()
        pltpu.make_async_copy(jj_h.at[pl.ds(row0, ROWS)], jj_v.at[s4],
                              in_sem.at[s4]).wait()
        pltpu.make_async_copy(rr_h.at[pl.ds(row0, ROWS)], r_v.at[s4],
                              in_sem.at[s4]).wait()
        pltpu.make_async_copy(cc_h.at[pl.ds(row0, ROWS)], cu_v.at[s4],
                              in_sem.at[s4]).wait()

    def _drain_sc(b):
        s3 = lax.rem(b, 3)
        s5 = lax.rem(b, 5)
        for c in range(ROWS):
            pltpu.make_async_copy(ct_v.at[s3, c],
                                  e_sh.at[ii_v.at[s5, c]],
                                  sc_sem.at[s3]).wait()

    for pb in range(3):
        @pl.when(pb < nb)
        def _():
            _issue_in(pb)

    def _block(b, carry):
        s3 = lax.rem(b, 3)
        s2 = lax.rem(b, 2)
        s5 = lax.rem(b, 5)
        s4 = lax.rem(b, 4)

        @pl.when(b >= 2)
        def _():
            _drain_sc(b - 2)

        @pl.when(b + 3 < nb)
        def _():
            _issue_in(b + 3)

        _wait_in(b)

        @plsc.parallel_loop(0, ROWS * 8, unroll=2)
        def _row(v):
            c = lax.shift_right_logical(v, 3)
            q = jnp.bitwise_and(v, 7)
            if True:
                sl = pl.ds(q * LANES, LANES)
                iiv = ii_v[s5, c, sl]
                jjv = jj_v[s4, c, sl]
                wi = plsc.load_gather(tab_v, [iiv])
                wj = plsc.load_gather(tab_v, [jjv])
                zi = _dec(lax.shift_right_logical(wi, 16))
                zj = _dec(lax.shift_right_logical(wj, 16))
                qi = _dec(jnp.bitwise_and(wi, 0xFFFF))
                qj = _dec(jnp.bitwise_and(wj, 0xFFFF))
                rv = r_v[s4, c, sl]
                cuv = cu_v[s4, c, sl]
                s = (zi + zj) * rv
                f = (cc1 * jnp.exp(nq1 * s) + cc2 * jnp.exp(nq2 * s)
                     + cc3 * jnp.exp(nq3 * s) + cc4 * jnp.exp(nq4 * s))
                ct_v[s3, c, sl] = f * (qi * qj) * (cuv / rv)

        for c in range(ROWS):
            pltpu.async_copy(ct_v.at[s3, c], e_sh.at[ii_v.at[s5, c]],
                             sc_sem.at[s3], add=True)
        return carry

    lax.fori_loop(0, nb, _block, 0)

    @pl.when(nb >= 2)
    def _():
        _drain_sc(nb - 2)

    @pl.when(nb >= 1)
    def _():
        _drain_sc(nb - 1)

    plsc.subcore_barrier()

    @pl.when(sid == 0)
    def _():
        pltpu.sync_copy(e_sh, out_h.at[cid])


def kernel(N, Zf, rij, cutoff_values, idx_i, idx_j,
           _adiv, _apow, _c1, _c2, _c3, _c4, _a1, _a2, _a3, _a4):
    f32 = jnp.float32
    sp = jax.nn.softplus
    n = Zf.shape[0]
    p = rij.shape[0]
    npad = ((n + ZCH - 1) // ZCH) * ZCH
    nchunks = npad // ZCH
    nrows = p // CHUNK
    tblk = nrows // ROWS
    nblk_base, nblk_extra = divmod(tblk, NW)

    adiv = sp(_adiv)[0]
    apow = sp(_apow)[0]
    aa = [sp(x)[0] for x in (_a1, _a2, _a3, _a4)]
    cc = [sp(x)[0] for x in (_c1, _c2, _c3, _c4)]
    csum = cc[0] + cc[1] + cc[2] + cc[3]
    keh = KE / 2.0
    par = jnp.stack([-(aa[0] * adiv), -(aa[1] * adiv),
                     -(aa[2] * adiv), -(aa[3] * adiv),
                     keh * cc[0] / csum, keh * cc[1] / csum,
                     keh * cc[2] / csum, keh * cc[3] / csum]).astype(f32)
    par = jnp.broadcast_to(par[:, None], (8, LANES))

    zf_pad = jnp.concatenate([Zf.astype(f32),
                              jnp.ones((npad - n,), f32)])
    zf2 = zf_pad.reshape(npad // CHUNK, CHUNK)
    ap1 = jnp.reshape(apow, (1,)).astype(f32)

    tab2 = pl.pallas_call(
        _table_body,
        out_shape=jax.ShapeDtypeStruct((npad // CHUNK, CHUNK), jnp.int32),
        in_specs=[pl.BlockSpec(memory_space=pltpu.VMEM),
                  pl.BlockSpec(memory_space=pltpu.SMEM)],
        out_specs=pl.BlockSpec(memory_space=pltpu.VMEM),
    )(zf2, ap1)
    tab = tab2.reshape(npad)

    ii2 = idx_i.astype(jnp.int32).reshape(nrows, CHUNK)
    jj2 = idx_j.astype(jnp.int32).reshape(nrows, CHUNK)
    rr2 = rij.astype(f32).reshape(nrows, CHUNK)
    cc2 = cutoff_values.astype(f32).reshape(nrows, CHUNK)

    mesh = plsc.VectorSubcoreMesh(core_axis_name="c", subcore_axis_name="s")
    parts = pl.kernel(
        functools.partial(_sc_body, nblk_base, nblk_extra, nchunks),
        out_type=jax.ShapeDtypeStruct((NC, npad), f32),
        mesh=mesh,
        compiler_params=pltpu.CompilerParams(needs_layout_passes=False),
        scratch_types=[
            pltpu.VMEM((npad,), jnp.int32),
            pltpu.VMEM((8, LANES), f32),
            pltpu.VMEM((ZCH,), f32),
            pltpu.VMEM((5, ROWS, CHUNK), jnp.int32),
            pltpu.VMEM((4, ROWS, CHUNK), jnp.int32),
            pltpu.VMEM((4, ROWS, CHUNK), f32),
            pltpu.VMEM((4, ROWS, CHUNK), f32),
            pltpu.VMEM((3, ROWS, CHUNK), f32),
            pltpu.VMEM_SHARED((npad,), f32),
            pltpu.SemaphoreType.DMA((5,)),
            pltpu.SemaphoreType.DMA((4,)),
            pltpu.SemaphoreType.DMA((3,)),
            pltpu.SemaphoreType.DMA,
        ],
    )(tab, par, ii2, jj2, rr2, cc2)

    e2 = pl.pallas_call(
        _sum_body,
        out_shape=jax.ShapeDtypeStruct((npad // CHUNK, CHUNK), f32),
        in_specs=[pl.BlockSpec(memory_space=pltpu.VMEM)],
        out_specs=pl.BlockSpec(memory_space=pltpu.VMEM),
    )(parts.reshape(NC, npad // CHUNK, CHUNK))
    return e2.reshape(npad)[:n]


# speedup vs baseline: 1.1438x; 1.0002x over previous
import functools

import jax
import jax.numpy as jnp
from jax import lax
from jax.experimental import pallas as pl
from jax.experimental.pallas import tpu as pltpu
from jax.experimental.pallas import tpu_sc as plsc

KE = 14.399645351950548

NC = 2
NS = 16
NW = NC * NS
LANES = 16
CHUNK = 128
ROWS = 8
BLK = ROWS * CHUNK
ZCH = 1024


def _f16bits(u):
    rnd = jnp.bitwise_and(lax.shift_right_logical(u, 13), 1)
    return lax.shift_right_logical(u + 0xFFF + rnd - 0x38000000, 13)


def _table_body(zf_ref, ap_ref, out_ref):
    zf = zf_ref[...]
    z = jnp.exp(ap_ref[0] * jnp.log(zf))
    hz = _f16bits(lax.bitcast_convert_type(z, jnp.int32))
    hq = _f16bits(lax.bitcast_convert_type(zf, jnp.int32))
    out_ref[...] = jnp.bitwise_or(lax.shift_left(hz, 16), hq)


def _sum_body(p_ref, o_ref):
    o_ref[...] = p_ref[0] + p_ref[1]


def _dec(h):
    return lax.bitcast_convert_type(lax.shift_left(h, 13) + 0x38000000,
                                    jnp.float32)


def _sc_body(nblk_base, nblk_extra, nchunks,
             tab_h, par_h, zer_h, ii_h, jj_h, rr_h, cc_h, out_h,
             tab_v, par_v, ii_v, jj_v, r_v, cu_v, ct_v, e_sh,
             ii_sem, in_sem, sc_sem, aux_sem):
    cid = lax.axis_index("c")
    sid = lax.axis_index("s")
    wid = sid * NC + cid

    nb = nblk_base + (wid < nblk_extra).astype(jnp.int32)
    sb = wid * nblk_base + jnp.minimum(wid, nblk_extra)

    tcp = pltpu.async_copy(tab_h, tab_v, aux_sem)
    pcp = pltpu.async_copy(par_h, par_v, aux_sem)

    for k in range((nchunks + NS - 1) // NS):
        ci = sid + k * NS

        @pl.when(ci < nchunks)
        def _():
            pltpu.sync_copy(zer_h.at[pl.ds(ci * ZCH, ZCH)],
                            e_sh.at[pl.ds(ci * ZCH, ZCH)])

    tcp.wait()
    pcp.wait()
    plsc.subcore_barrier()

    nq1 = par_v[0]
    nq2 = par_v[1]
    nq3 = par_v[2]
    nq4 = par_v[3]
    cc1 = par_v[4]
    cc2 = par_v[5]
    cc3 = par_v[6]
    cc4 = par_v[7]

    def _issue_in(b):
        row0 = (sb + b) * ROWS
        s5 = lax.rem(b, 6)
        s4 = lax.rem(b, 5)
        pltpu.async_copy(ii_h.at[pl.ds(row0, ROWS)], ii_v.at[s5],
                         ii_sem.at[s5])
        pltpu.async_copy(jj_h.at[pl.ds(row0, ROWS)], jj_v.at[s4],
                         in_sem.at[s4])
        pltpu.async_copy(rr_h.at[pl.ds(row0, ROWS)], r_v.at[s4],
                         in_sem.at[s4])
        pltpu.async_copy(cc_h.at[pl.ds(row0, ROWS)], cu_v.at[s4],
                         in_sem.at[s4])

    def _wait_in(b):
        row0 = (sb + b) * ROWS
        s5 = lax.rem(b, 6)
        s4 = lax.rem(b, 5)
        pltpu.make_async_copy(ii_h.at[pl.ds(row0, ROWS)], ii_v.at[s5],
                              ii_sem.at[s5]).wait()
        pltpu.make_async_copy(jj_h.at[pl.ds(row0, ROWS)], jj_v.at[s4],
                              in_sem.at[s4]).wait()
        pltpu.make_async_copy(rr_h.at[pl.ds(row0, ROWS)], r_v.at[s4],
                              in_sem.at[s4]).wait()
        pltpu.make_async_copy(cc_h.at[pl.ds(row0, ROWS)], cu_v.at[s4],
                              in_sem.at[s4]).wait()

    def _drain_sc(b):
        s3 = lax.rem(b, 2)
        s5 = lax.rem(b, 6)
        for c in range(ROWS):
            pltpu.make_async_copy(ct_v.at[s3, c],
                                  e_sh.at[ii_v.at[s5, c]],
                                  sc_sem.at[s3]).wait()

    for pb in range(4):
        @pl.when(pb < nb)
        def _():
            _issue_in(pb)

    def _block(b, carry):
        s3 = lax.rem(b, 2)
        s5 = lax.rem(b, 6)
        s4 = lax.rem(b, 5)

        @pl.when(b >= 2)
        def _():
            _drain_sc(b - 2)

        @pl.when(b + 4 < nb)
        def _():
            _issue_in(b + 4)

        _wait_in(b)

        @plsc.parallel_loop(0, ROWS * 8, unroll=2)
        def _row(v):
            c = lax.shift_right_logical(v, 3)
            q = jnp.bitwise_and(v, 7)
            if True:
                sl = pl.ds(q * LANES, LANES)
                iiv = ii_v[s5, c, sl]
                jjv = jj_v[s4, c, sl]
                wi = plsc.load_gather(tab_v, [iiv])
                wj = plsc.load_gather(tab_v, [jjv])
                zi = _dec(lax.shift_right_logical(wi, 16))
                zj = _dec(lax.shift_right_logical(wj, 16))
                qi = _dec(jnp.bitwise_and(wi, 0xFFFF))
                qj = _dec(jnp.bitwise_and(wj, 0xFFFF))
                rv = r_v[s4, c, sl]
                cuv = cu_v[s4, c, sl]
                s = (zi + zj) * rv
                f = (cc1 * jnp.exp(nq1 * s) + cc2 * jnp.exp(nq2 * s)
                     + cc3 * jnp.exp(nq3 * s) + cc4 * jnp.exp(nq4 * s))
                ct_v[s3, c, sl] = f * (qi * qj) * (cuv / rv)

        for c in range(ROWS):
            pltpu.async_copy(ct_v.at[s3, c], e_sh.at[ii_v.at[s5, c]],
                             sc_sem.at[s3], add=True)
        return carry

    lax.fori_loop(0, nb, _block, 0)

    @pl.when(nb >= 2)
    def _():
        _drain_sc(nb - 2)

    @pl.when(nb >= 1)
    def _():
        _drain_sc(nb - 1)

    plsc.subcore_barrier()

    @pl.when(sid == 0)
    def _():
        pltpu.sync_copy(e_sh, out_h.at[cid])


def kernel(N, Zf, rij, cutoff_values, idx_i, idx_j,
           _adiv, _apow, _c1, _c2, _c3, _c4, _a1, _a2, _a3, _a4):
    f32 = jnp.float32
    sp = jax.nn.softplus
    n = Zf.shape[0]
    p = rij.shape[0]
    npad = ((n + ZCH - 1) // ZCH) * ZCH
    nchunks = npad // ZCH
    nrows = p // CHUNK
    tblk = nrows // ROWS
    nblk_base, nblk_extra = divmod(tblk, NW)

    adiv = sp(_adiv)[0]
    apow = sp(_apow)[0]
    aa = [sp(x)[0] for x in (_a1, _a2, _a3, _a4)]
    cc = [sp(x)[0] for x in (_c1, _c2, _c3, _c4)]
    csum = cc[0] + cc[1] + cc[2] + cc[3]
    keh = KE / 2.0
    par = jnp.stack([-(aa[0] * adiv), -(aa[1] * adiv),
                     -(aa[2] * adiv), -(aa[3] * adiv),
                     keh * cc[0] / csum, keh * cc[1] / csum,
                     keh * cc[2] / csum, keh * cc[3] / csum]).astype(f32)
    par = jnp.broadcast_to(par[:, None], (8, LANES))

    zf_pad = jnp.concatenate([Zf.astype(f32),
                              jnp.ones((npad - n,), f32)])
    zf2 = zf_pad.reshape(npad // CHUNK, CHUNK)
    ap1 = jnp.reshape(apow, (1,)).astype(f32)

    tab2 = pl.pallas_call(
        _table_body,
        out_shape=jax.ShapeDtypeStruct((npad // CHUNK, CHUNK), jnp.int32),
        in_specs=[pl.BlockSpec(memory_space=pltpu.VMEM),
                  pl.BlockSpec(memory_space=pltpu.SMEM)],
        out_specs=pl.BlockSpec(memory_space=pltpu.VMEM),
    )(zf2, ap1)
    tab = tab2.reshape(npad)[:n]
    zer = jnp.zeros((npad,), f32)

    ii2 = idx_i.astype(jnp.int32).reshape(nrows, CHUNK)
    jj2 = idx_j.astype(jnp.int32).reshape(nrows, CHUNK)
    rr2 = rij.astype(f32).reshape(nrows, CHUNK)
    cc2 = cutoff_values.astype(f32).reshape(nrows, CHUNK)

    mesh = plsc.VectorSubcoreMesh(core_axis_name="c", subcore_axis_name="s")
    parts = pl.kernel(
        functools.partial(_sc_body, nblk_base, nblk_extra, nchunks),
        out_type=jax.ShapeDtypeStruct((NC, npad), f32),
        mesh=mesh,
        compiler_params=pltpu.CompilerParams(needs_layout_passes=False),
        scratch_types=[
            pltpu.VMEM((n,), jnp.int32),
            pltpu.VMEM((8, LANES), f32),
            pltpu.VMEM((6, ROWS, CHUNK), jnp.int32),
            pltpu.VMEM((5, ROWS, CHUNK), jnp.int32),
            pltpu.VMEM((5, ROWS, CHUNK), f32),
            pltpu.VMEM((5, ROWS, CHUNK), f32),
            pltpu.VMEM((2, ROWS, CHUNK), f32),
            pltpu.VMEM_SHARED((npad,), f32),
            pltpu.SemaphoreType.DMA((6,)),
            pltpu.SemaphoreType.DMA((5,)),
            pltpu.SemaphoreType.DMA((2,)),
            pltpu.SemaphoreType.DMA,
        ],
    )(tab, par, zer, ii2, jj2, rr2, cc2)

    e2 = pl.pallas_call(
        _sum_body,
        out_shape=jax.ShapeDtypeStruct((npad // CHUNK, CHUNK), f32),
        in_specs=[pl.BlockSpec(memory_space=pltpu.VMEM)],
        out_specs=pl.BlockSpec(memory_space=pltpu.VMEM),
    )(parts.reshape(NC, npad // CHUNK, CHUNK))
    return e2.reshape(npad)[:n]
